# baseline (speedup 1.0000x reference)
import jax
import jax.numpy as jnp
from jax import lax
from jax.experimental import pallas as pl
from jax.experimental.pallas import tpu as pltpu
from jax.experimental.pallas import tpu_sc as plsc

N, C, H, W = 4, 96, 512, 512
HW = H * W
P = N * HW
L = 16
NC, NS = 2, 16
NW = NC * NS
CHUNK = 2048
NCHUNK = 13
GROUPS = CHUNK // L
PW = NCHUNK * CHUNK
IMGW = 8
SC_PLANE = IMGW * PW
TCR = HW - SC_PLANE
TCROWS = 128
TC_SUB = TCR // (TCROWS * 128)


def _sc_body(pred_ref, tgt_ref, out_ref,
             t_v, idx0, idx1, rows0, rows1, acc_v, sem0, sem1):
    cid = lax.axis_index("c")
    sid = lax.axis_index("s")
    wid = sid * NC + cid
    n = wid // IMGW
    plane0 = (wid % IMGW) * PW

    pltpu.sync_copy(tgt_ref.at[pl.ds(n * HW + plane0, PW)], t_v)

    iota = lax.iota(jnp.int32, L)
    ebase = n * C * HW + plane0

    def build_idx(c, idx_ref):
        def g_body(j, _):
            for u in range(4):
                o = (j * 4 + u) * L
                tv = t_v[pl.ds(c * CHUNK + o, L)]
                elems = tv * HW + (ebase + c * CHUNK + o) + iota
                idx_ref[pl.ds(o, L)] = elems
            return 0
        lax.fori_loop(0, GROUPS // 4, g_body, 0)

    def extract(rows_ref, acc):
        def e_body(j, acc):
            for u in range(4):
                o = (j * 4 + u) * L
                acc = acc + rows_ref[pl.ds(o, L)]
            return acc
        return lax.fori_loop(0, GROUPS // 4, e_body, acc)

    idxs = [idx0, idx1]
    rows = [rows0, rows1]
    sems = [sem0, sem1]
    copies = {}

    build_idx(0, idxs[0])
    copies[0] = pltpu.async_copy(pred_ref.at[idxs[0]], rows[0], sems[0])

    acc = jnp.zeros((L,), jnp.float32)
    for c in range(NCHUNK):
        cur = c % 2
        if c + 1 < NCHUNK:
            nxt = (c + 1) % 2
            build_idx(c + 1, idxs[nxt])
            copies[c + 1] = pltpu.async_copy(
                pred_ref.at[idxs[nxt]], rows[nxt], sems[nxt])
        copies[c].wait()
        acc = extract(rows[cur], acc)

    acc_v[...] = acc
    pltpu.sync_copy(acc_v, out_ref.at[pl.ds(wid * L, L)])


def _tc_body(pred_blk, tgt_blk, out_blk):
    n_ = pl.program_id(0)
    s_ = pl.program_id(1)

    @pl.when((n_ == 0) & (s_ == 0))
    def _():
        out_blk[...] = jnp.zeros_like(out_blk)

    t = tgt_blk[0].reshape(TCROWS // 8, 8, 128)
    acc = jnp.zeros((8, 128), jnp.float32)
    for c in range(C):
        p = pred_blk[c].reshape(TCROWS // 8, 8, 128)
        acc = acc + jnp.sum(jnp.where(t == c, p, 0.0), axis=0)
    out_blk[...] += acc


@jax.jit
def _nll_partials(table, tgt):
    mesh = plsc.VectorSubcoreMesh(core_axis_name="c", subcore_axis_name="s")
    sc = pl.kernel(
        _sc_body,
        out_type=jax.ShapeDtypeStruct((NW * L,), jnp.float32),
        mesh=mesh,
        scratch_types=[
            pltpu.VMEM((PW,), jnp.int32),
            pltpu.VMEM((CHUNK,), jnp.int32),
            pltpu.VMEM((CHUNK,), jnp.int32),
            pltpu.VMEM((CHUNK,), jnp.float32),
            pltpu.VMEM((CHUNK,), jnp.float32),
            pltpu.VMEM((L,), jnp.float32),
            pltpu.SemaphoreType.DMA,
            pltpu.SemaphoreType.DMA,
        ],
    )
    sc_partials = sc(table, tgt)

    table3 = table.reshape(N * C, HW // 128, 128)
    tgt3 = tgt.reshape(N, HW // 128, 128)
    tc_partials = pl.pallas_call(
        _tc_body,
        grid=(N, TC_SUB),
        in_specs=[
            pl.BlockSpec((C, TCROWS, 128),
                         lambda n_, s_: (n_, SC_PLANE // 128 // TCROWS + s_,
                                         0)),
            pl.BlockSpec((1, TCROWS, 128),
                         lambda n_, s_: (n_, SC_PLANE // 128 // TCROWS + s_,
                                         0)),
        ],
        out_specs=pl.BlockSpec((8, 128), lambda n_, s_: (0, 0)),
        out_shape=jax.ShapeDtypeStruct((8, 128), jnp.float32),
    )(table3, tgt3)

    return sc_partials, tc_partials


def kernel(pred, target):
    table = (pred.reshape(N, C, H // 8, 8, W // 128, 128)
             .transpose(0, 1, 2, 4, 3, 5)
             .reshape(-1))
    tgt = (target.astype(jnp.int32)
           .reshape(N, H // 8, 8, W // 128, 128)
           .transpose(0, 1, 3, 2, 4)
           .reshape(-1))
    sc_partials, tc_partials = _nll_partials(table, tgt)
    return -(jnp.sum(sc_partials) + jnp.sum(tc_partials)) / P


# speedup vs baseline: 1.0666x; 1.0666x over previous
import jax
import jax.numpy as jnp
from jax import lax
from jax.experimental import pallas as pl
from jax.experimental.pallas import tpu as pltpu
from jax.experimental.pallas import tpu_sc as plsc

N, C, H, W = 4, 96, 512, 512
HW = H * W
P = N * HW
L = 16
NC, NS = 2, 16
NW = NC * NS
PW = P // NW
CHUNK = 2048
NCHUNK = PW // CHUNK
GROUPS = CHUNK // L
ROW_T = HW // L
ROW_N = C * HW // L
IMGW = HW // PW


def _sc_body(pred_ref, tgt_ref, out_ref,
             t_v, idx0, idx1, rows0, rows1, acc_v, sem0, sem1):
    cid = lax.axis_index("c")
    sid = lax.axis_index("s")
    wid = sid * NC + cid
    n = wid // IMGW

    pltpu.sync_copy(tgt_ref.at[pl.ds(wid * PW, PW)], t_v)

    iota = lax.iota(jnp.int32, L)

    ebase = wid * PW + n * (C - 1) * HW

    def build_idx(c, idx_ref):
        def g_body(j, _):
            for u in range(4):
                o = (j * 4 + u) * L
                tv = t_v[pl.ds(c * CHUNK + o, L)]
                elems = tv * HW + (ebase + c * CHUNK + o) + iota
                idx_ref[pl.ds(o, L)] = elems
            return 0
        lax.fori_loop(0, GROUPS // 4, g_body, 0)

    def extract(rows_ref, acc):
        def e_body(j, acc):
            for u in range(4):
                g = j * 4 + u
                acc = acc + rows_ref[pl.ds(g * L, L)]
            return acc
        return lax.fori_loop(0, GROUPS // 4, e_body, acc)

    idxs = [idx0, idx1]
    rows = [rows0, rows1]
    sems = [sem0, sem1]
    copies = {}

    build_idx(0, idxs[0])
    copies[0] = pltpu.async_copy(pred_ref.at[idxs[0]], rows[0], sems[0])

    acc = jnp.zeros((L,), jnp.float32)
    for c in range(NCHUNK):
        cur = c % 2
        if c + 1 < NCHUNK:
            nxt = (c + 1) % 2
            build_idx(c + 1, idxs[nxt])
            copies[c + 1] = pltpu.async_copy(
                pred_ref.at[idxs[nxt]], rows[nxt], sems[nxt])
        copies[c].wait()
        acc = extract(rows[cur], acc)

    acc_v[...] = acc
    pltpu.sync_copy(acc_v, out_ref.at[pl.ds(wid * L, L)])


@jax.jit
def _nll_sum(table, tgt):
    mesh = plsc.VectorSubcoreMesh(core_axis_name="c", subcore_axis_name="s")
    f = pl.kernel(
        _sc_body,
        out_type=jax.ShapeDtypeStruct((NW * L,), jnp.float32),
        mesh=mesh,
        scratch_types=[
            pltpu.VMEM((PW,), jnp.int32),
            pltpu.VMEM((CHUNK,), jnp.int32),
            pltpu.VMEM((CHUNK,), jnp.int32),
            pltpu.VMEM((CHUNK,), jnp.float32),
            pltpu.VMEM((CHUNK,), jnp.float32),
            pltpu.VMEM((L,), jnp.float32),
            pltpu.SemaphoreType.DMA,
            pltpu.SemaphoreType.DMA,
        ],
    )
    return f(table, tgt)


def kernel(pred, target):
    table = (pred.reshape(N, C, H // 8, 8, W // 128, 128)
             .transpose(0, 1, 2, 4, 3, 5)
             .reshape(-1))
    tgt = (target.astype(jnp.int32)
           .reshape(N, H // 8, 8, W // 128, 128)
           .transpose(0, 1, 3, 2, 4)
           .reshape(-1))
    partials = _nll_sum(table, tgt)
    return -jnp.sum(partials) / P


# speedup vs baseline: 1.0785x; 1.0111x over previous
import jax
import jax.numpy as jnp
from jax import lax
from jax.experimental import pallas as pl
from jax.experimental.pallas import tpu as pltpu
from jax.experimental.pallas import tpu_sc as plsc

N, C, H, W = 4, 96, 512, 512
HW = H * W
P = N * HW
L = 16
NC, NS = 2, 16
NW = NC * NS
PW = P // NW
CHUNK = 4096
NCHUNK = PW // CHUNK
GROUPS = CHUNK // L
ROW_T = HW // L
ROW_N = C * HW // L
IMGW = HW // PW


def _sc_body(pred_ref, tgt_ref, out_ref,
             t_v, idx0, idx1, rows0, rows1, acc_v, sem0, sem1):
    cid = lax.axis_index("c")
    sid = lax.axis_index("s")
    wid = sid * NC + cid
    n = wid // IMGW

    pltpu.sync_copy(tgt_ref.at[pl.ds(wid * PW, PW)], t_v)

    iota = lax.iota(jnp.int32, L)

    ebase = wid * PW + n * (C - 1) * HW

    def build_idx(c, idx_ref):
        def g_body(j, _):
            for u in range(4):
                o = (j * 4 + u) * L
                tv = t_v[pl.ds(c * CHUNK + o, L)]
                elems = tv * HW + (ebase + c * CHUNK + o) + iota
                idx_ref[pl.ds(o, L)] = elems
            return 0
        lax.fori_loop(0, GROUPS // 4, g_body, 0)

    def extract(rows_ref, acc):
        def e_body(j, acc):
            for u in range(4):
                g = j * 4 + u
                acc = acc + rows_ref[pl.ds(g * L, L)]
            return acc
        return lax.fori_loop(0, GROUPS // 4, e_body, acc)

    idxs = [idx0, idx1]
    rows = [rows0, rows1]
    sems = [sem0, sem1]
    copies = {}

    build_idx(0, idxs[0])
    copies[0] = pltpu.async_copy(pred_ref.at[idxs[0]], rows[0], sems[0])

    acc = jnp.zeros((L,), jnp.float32)
    for c in range(NCHUNK):
        cur = c % 2
        if c + 1 < NCHUNK:
            nxt = (c + 1) % 2
            build_idx(c + 1, idxs[nxt])
            copies[c + 1] = pltpu.async_copy(
                pred_ref.at[idxs[nxt]], rows[nxt], sems[nxt])
        copies[c].wait()
        acc = extract(rows[cur], acc)

    acc_v[...] = acc
    pltpu.sync_copy(acc_v, out_ref.at[pl.ds(wid * L, L)])


@jax.jit
def _nll_sum(table, tgt):
    mesh = plsc.VectorSubcoreMesh(core_axis_name="c", subcore_axis_name="s")
    f = pl.kernel(
        _sc_body,
        out_type=jax.ShapeDtypeStruct((NW * L,), jnp.float32),
        mesh=mesh,
        scratch_types=[
            pltpu.VMEM((PW,), jnp.int32),
            pltpu.VMEM((CHUNK,), jnp.int32),
            pltpu.VMEM((CHUNK,), jnp.int32),
            pltpu.VMEM((CHUNK,), jnp.float32),
            pltpu.VMEM((CHUNK,), jnp.float32),
            pltpu.VMEM((L,), jnp.float32),
            pltpu.SemaphoreType.DMA,
            pltpu.SemaphoreType.DMA,
        ],
    )
    return f(table, tgt)


def kernel(pred, target):
    table = (pred.reshape(N, C, H // 8, 8, W // 128, 128)
             .transpose(0, 1, 2, 4, 3, 5)
             .reshape(-1))
    tgt = (target.astype(jnp.int32)
           .reshape(N, H // 8, 8, W // 128, 128)
           .transpose(0, 1, 3, 2, 4)
           .reshape(-1))
    partials = _nll_sum(table, tgt)
    return -jnp.sum(partials) / P
